# baseline (speedup 1.0000x reference)
import jax
import jax.numpy as jnp
from jax.experimental import pallas as pl
from jax.experimental.pallas import tpu as pltpu

_TN = 1024
_NBUF = 6


def _mlp_body(idx_hbm, emb_hbm, w1_hbm, b1_hbm, b2_hbm, w2_hbm, out_hbm,
              idx_ref, emb_ref, w1_ref, b1_ref, b2_ref,
              htab_ref, oh_ref, w2buf, outbuf, rsem, wsem, isem):
    T, V = idx_ref.shape[0], emb_ref.shape[0]
    N = w2_hbm.shape[1]
    steps = N // _TN

    def read_start(j, slot):
        pltpu.make_async_copy(
            w2_hbm.at[:, pl.ds(j * _TN, _TN)], w2buf.at[slot], rsem.at[slot]
        ).start()

    def read_wait(j, slot):
        pltpu.make_async_copy(
            w2_hbm.at[:, pl.ds(j * _TN, _TN)], w2buf.at[slot], rsem.at[slot]
        ).wait()

    for p in range(_NBUF):
        read_start(p, p)

    small = [(idx_hbm, idx_ref), (emb_hbm, emb_ref), (w1_hbm, w1_ref),
             (b1_hbm, b1_ref), (b2_hbm, b2_ref)]
    for k, (src, dst) in enumerate(small):
        pltpu.make_async_copy(src, dst, isem.at[k]).start()
    for k, (src, dst) in enumerate(small):
        pltpu.make_async_copy(src, dst, isem.at[k]).wait()

    h = jnp.dot(emb_ref[...], w1_ref[...], preferred_element_type=jnp.float32)
    htab_ref[...] = jnp.tanh(h + b1_ref[...])
    iota = jax.lax.broadcasted_iota(jnp.int32, (T, V), 1)
    oh_ref[...] = jnp.where(iota == idx_ref[...], 1.0, 0.0).astype(jnp.float32)

    for j in range(steps):
        slot = j % _NBUF
        read_wait(j, slot)
        m = jnp.dot(htab_ref[...], w2buf[slot], preferred_element_type=jnp.float32)
        o = (
            jnp.dot(oh_ref[...], m, preferred_element_type=jnp.float32)
            + b2_ref[:, j * _TN:(j + 1) * _TN]
        )
        if j >= _NBUF:
            pltpu.make_async_copy(
                outbuf.at[slot],
                out_hbm.at[:, pl.ds((j - _NBUF) * _TN, _TN)],
                wsem.at[slot],
            ).wait()
        outbuf[slot] = o
        pltpu.make_async_copy(
            outbuf.at[slot], out_hbm.at[:, pl.ds(j * _TN, _TN)], wsem.at[slot]
        ).start()
        if j + _NBUF < steps:
            read_start(j + _NBUF, slot)

    for j in range(max(0, steps - _NBUF), steps):
        slot = j % _NBUF
        pltpu.make_async_copy(
            outbuf.at[slot], out_hbm.at[:, pl.ds(j * _TN, _TN)], wsem.at[slot]
        ).wait()


def kernel(prefix, emb, W1, b1, W2, b2):
    B, L = prefix.shape
    V, D = emb.shape
    H = W1.shape[1]
    N = W2.shape[1]
    T = B * L

    idx = prefix.reshape(T, 1).astype(jnp.int32)

    out = pl.pallas_call(
        _mlp_body,
        in_specs=[pl.BlockSpec(memory_space=pl.ANY)] * 6,
        out_specs=pl.BlockSpec(memory_space=pl.ANY),
        out_shape=jax.ShapeDtypeStruct((T, N), jnp.float32),
        scratch_shapes=[
            pltpu.VMEM((T, 1), jnp.int32),
            pltpu.VMEM((V, D), jnp.float32),
            pltpu.VMEM((D, H), jnp.float32),
            pltpu.VMEM((1, H), jnp.float32),
            pltpu.VMEM((1, N), jnp.float32),
            pltpu.VMEM((V, H), jnp.float32),
            pltpu.VMEM((T, V), jnp.float32),
            pltpu.VMEM((_NBUF, H, _TN), jnp.float32),
            pltpu.VMEM((_NBUF, T, _TN), jnp.float32),
            pltpu.SemaphoreType.DMA((_NBUF,)),
            pltpu.SemaphoreType.DMA((_NBUF,)),
            pltpu.SemaphoreType.DMA((5,)),
        ],
    )(idx, emb, W1, b1.reshape(1, H), b2.reshape(1, N), W2)

    return out.reshape(B, L, N)


# speedup vs baseline: 1.0086x; 1.0086x over previous
import jax
import jax.numpy as jnp
from jax.experimental import pallas as pl
from jax.experimental.pallas import tpu as pltpu

_TN = 1024
_NBUF = 6


def _mlp_body(idx_hbm, emb_hbm, w1_hbm, b1_hbm, b2_hbm, w2_hbm, out_hbm,
              idx_ref, emb_ref, w1_ref, b1_ref, b2_ref,
              htab_ref, oh_ref, w2buf, outbuf, rsem, wsem, isem):
    T, V = idx_ref.shape[0], emb_ref.shape[0]
    N = w2_hbm.shape[1]
    steps = N // _TN

    def read_start(j, slot):
        pltpu.make_async_copy(
            w2_hbm.at[:, pl.ds(j * _TN, _TN)], w2buf.at[slot], rsem.at[slot]
        ).start()

    def read_wait(j, slot):
        pltpu.make_async_copy(
            w2_hbm.at[:, pl.ds(j * _TN, _TN)], w2buf.at[slot], rsem.at[slot]
        ).wait()

    small = [(idx_hbm, idx_ref), (emb_hbm, emb_ref), (w1_hbm, w1_ref),
             (b1_hbm, b1_ref), (b2_hbm, b2_ref)]
    for k, (src, dst) in enumerate(small):
        pltpu.make_async_copy(src, dst, isem.at[k]).start()

    for p in range(_NBUF):
        read_start(p, p)

    for k, (src, dst) in enumerate(small):
        pltpu.make_async_copy(src, dst, isem.at[k]).wait()

    h = jnp.dot(emb_ref[...], w1_ref[...], preferred_element_type=jnp.float32)
    htab_ref[...] = jnp.tanh(h + b1_ref[...])
    iota = jax.lax.broadcasted_iota(jnp.int32, (T, V), 1)
    oh_ref[...] = jnp.where(iota == idx_ref[...], 1.0, 0.0).astype(jnp.float32)

    for j in range(steps):
        slot = j % _NBUF
        read_wait(j, slot)
        m = jnp.dot(htab_ref[...], w2buf[slot], preferred_element_type=jnp.float32)
        o = (
            jnp.dot(oh_ref[...], m, preferred_element_type=jnp.float32)
            + b2_ref[:, j * _TN:(j + 1) * _TN]
        )
        if j >= _NBUF:
            pltpu.make_async_copy(
                outbuf.at[slot],
                out_hbm.at[:, pl.ds((j - _NBUF) * _TN, _TN)],
                wsem.at[slot],
            ).wait()
        outbuf[slot] = o
        pltpu.make_async_copy(
            outbuf.at[slot], out_hbm.at[:, pl.ds(j * _TN, _TN)], wsem.at[slot]
        ).start()
        if j + _NBUF < steps:
            read_start(j + _NBUF, slot)

    for j in range(max(0, steps - _NBUF), steps):
        slot = j % _NBUF
        pltpu.make_async_copy(
            outbuf.at[slot], out_hbm.at[:, pl.ds(j * _TN, _TN)], wsem.at[slot]
        ).wait()


def kernel(prefix, emb, W1, b1, W2, b2):
    B, L = prefix.shape
    V, D = emb.shape
    H = W1.shape[1]
    N = W2.shape[1]
    T = B * L

    idx = prefix.reshape(T, 1).astype(jnp.int32)

    out = pl.pallas_call(
        _mlp_body,
        in_specs=[pl.BlockSpec(memory_space=pl.ANY)] * 6,
        out_specs=pl.BlockSpec(memory_space=pl.ANY),
        out_shape=jax.ShapeDtypeStruct((T, N), jnp.float32),
        scratch_shapes=[
            pltpu.VMEM((T, 1), jnp.int32),
            pltpu.VMEM((V, D), jnp.float32),
            pltpu.VMEM((D, H), jnp.float32),
            pltpu.VMEM((1, H), jnp.float32),
            pltpu.VMEM((1, N), jnp.float32),
            pltpu.VMEM((V, H), jnp.float32),
            pltpu.VMEM((T, V), jnp.float32),
            pltpu.VMEM((_NBUF, H, _TN), jnp.float32),
            pltpu.VMEM((_NBUF, T, _TN), jnp.float32),
            pltpu.SemaphoreType.DMA((_NBUF,)),
            pltpu.SemaphoreType.DMA((_NBUF,)),
            pltpu.SemaphoreType.DMA((5,)),
        ],
    )(idx, emb, W1, b1.reshape(1, H), b2.reshape(1, N), W2)

    return out.reshape(B, L, N)
